# baseline (device time: 73299 ns/iter reference)
import jax
import jax.numpy as jnp
from jax import lax
from jax.experimental import pallas as pl
from jax.experimental.pallas import tpu as pltpu

N_DEV = 4
BLK = 64
N_RES = 4
BF16 = jnp.bfloat16
QSCALE = 127.0 / 6.0

COMPUTE_ONLY = False
COMM_ONLY = False


def kernel(x, Wq, K_ext, V_ext, Wo):
    B, Sq_l, Dm = x.shape
    _, Skv_l, Hq, Dh = K_ext.shape
    HD = Hq * Dh
    n_blk = Sq_l // BLK
    blk_per_res = n_blk // N_RES
    R = blk_per_res * BLK
    n_hops = N_DEV - 1
    scale = 1.0 / (Dh ** 0.5)

    def quant(mat):
        return jnp.round(
            jnp.clip(mat * QSCALE, -127.0, 127.0)).astype(jnp.int8)

    def dequant(mat):
        return mat.astype(BF16) * BF16(1.0 / QSCALE)

    def res_rows(mat):
        out = []
        for r in range(N_RES):
            blocks = [r + N_RES * j for j in range(blk_per_res)]
            out.append(jnp.concatenate(
                [mat[rb * BLK:(rb + 1) * BLK] for rb in blocks], axis=0))
        return out

    def regroup(mat):
        return jnp.concatenate(res_rows(mat), axis=0)

    def to_heads(mat):
        return mat.reshape(mat.shape[0], Hq, Dh)

    def body(x_ref, wq_ref, k_ref, v_ref, wo_ref, out_ref,
             kvownA, kvownB, kvbufA, kvbufB,
             sA, rA, sB, rB):
        my = lax.axis_index("i")
        left = (my - 1) % N_DEV
        right = (my + 1) % N_DEV

        barrier_sem = pltpu.get_barrier_semaphore()
        for nbr in (left, right):
            pl.semaphore_signal(
                barrier_sem, inc=1,
                device_id=(nbr,), device_id_type=pl.DeviceIdType.MESH,
            )
        pl.semaphore_wait(barrier_sem, 2)

        kA = quant(regroup(k_ref[0].reshape(Skv_l, HD)))
        vA = quant(regroup(v_ref[0].reshape(Skv_l, HD)))
        kvownA[0, 0] = kA[:2 * R]
        kvownA[0, 1] = vA[:2 * R]
        kvownA[1, 0] = kA[2 * R:]
        kvownA[1, 1] = vA[2 * R:]

        def make_hop(h):
            common = dict(device_id_type=pl.DeviceIdType.MESH)
            subsA, subsB = [], []
            for s_ in range(2):
                subsA.append(pltpu.make_async_remote_copy(
                    src_ref=(kvownA if h == 0 else kvbufA.at[h - 1]).at[s_],
                    dst_ref=kvbufA.at[h, s_],
                    send_sem=sA.at[h, s_], recv_sem=rA.at[h, s_],
                    device_id=(right,), **common))
                subsB.append(pltpu.make_async_remote_copy(
                    src_ref=(kvownB if h == 0 else kvbufB.at[h - 1]).at[s_],
                    dst_ref=kvbufB.at[h, s_],
                    send_sem=sB.at[h, s_], recv_sem=rB.at[h, s_],
                    device_id=(left,), **common))
            return (subsA, subsB)

        if COMPUTE_ONLY:
            kB = quant(regroup(k_ref[1].reshape(Skv_l, HD)))
            vB = quant(regroup(v_ref[1].reshape(Skv_l, HD)))
            kvownB[0, 0] = kB[:2 * R]
            kvownB[0, 1] = vB[:2 * R]
            kvownB[1, 0] = kB[2 * R:]
            kvownB[1, 1] = vB[2 * R:]
            for h in range(n_hops):
                kvbufA[h] = kvownA[...]
                kvbufB[h] = kvownB[...]
            hops = []
        else:
            hops = [make_hop(h) for h in range(n_hops)]
            hops[0][0][0].start()
            hops[0][0][1].start()
            kB = quant(regroup(k_ref[1].reshape(Skv_l, HD)))
            vB = quant(regroup(v_ref[1].reshape(Skv_l, HD)))
            kvownB[0, 0] = kB[:2 * R]
            kvownB[0, 1] = vB[:2 * R]
            kvownB[1, 0] = kB[2 * R:]
            kvownB[1, 1] = vB[2 * R:]
            hops[0][1][0].start()
            hops[0][1][1].start()

        wq16 = wq_ref[...].astype(BF16)
        q16 = []
        for b in range(B):
            q_b = jnp.dot(x_ref[b].astype(BF16), wq16,
                          preferred_element_type=jnp.float32)
            q16.append([to_heads(qr.astype(BF16)) for qr in res_rows(q_b)])

        state = [[None] * N_RES for _ in range(B)]

        def process(b, k_rs, v_rs):
            for r in range(N_RES):
                q3 = q16[b][r]
                k3 = to_heads(k_rs[r])
                v3 = to_heads(v_rs[r])
                s = lax.dot_general(
                    q3, k3, (((2,), (2,)), ((1,), (1,))),
                    preferred_element_type=jnp.float32) * scale
                p = jnp.exp(s)
                l_c = jnp.sum(p, axis=-1, keepdims=True)
                acc_c = lax.dot_general(
                    p.astype(BF16), v3, (((2,), (0,)), ((0,), (1,))),
                    preferred_element_type=jnp.float32)
                st = state[b][r]
                state[b][r] = (
                    (l_c, acc_c) if st is None
                    else (st[0] + l_c, st[1] + acc_c))

        def fold_residue(b, r, k2, v2):
            q3 = q16[b][r]
            k3 = to_heads(k2)
            v3 = to_heads(v2)
            s = lax.dot_general(
                q3, k3, (((2,), (2,)), ((1,), (1,))),
                preferred_element_type=jnp.float32) * scale
            p = jnp.exp(s)
            l_c = jnp.sum(p, axis=-1, keepdims=True)
            acc_c = lax.dot_general(
                p.astype(BF16), v3, (((2,), (0,)), ((0,), (1,))),
                preferred_element_type=jnp.float32)
            st = state[b][r]
            state[b][r] = (
                (l_c, acc_c) if st is None
                else (st[0] + l_c, st[1] + acc_c))

        def process_sub(b, buf, h, s_):
            kv2 = dequant(buf[h, s_])
            for j in range(2):
                r = 2 * s_ + j
                fold_residue(b, r, kv2[0, j * R:(j + 1) * R],
                             kv2[1, j * R:(j + 1) * R])

        if not COMM_ONLY:
            process(0, res_rows(k_ref[0].reshape(Skv_l, HD).astype(BF16)),
                    res_rows(v_ref[0].reshape(Skv_l, HD).astype(BF16)))
            process(1, res_rows(k_ref[1].reshape(Skv_l, HD).astype(BF16)),
                    res_rows(v_ref[1].reshape(Skv_l, HD).astype(BF16)))

        for h in range(n_hops):
            for s_ in range(2):
                if not COMPUTE_ONLY:
                    hops[h][0][s_].wait_recv()
                    if h + 1 < n_hops:
                        hops[h + 1][0][s_].start()
                if not COMM_ONLY:
                    process_sub(0, kvbufA, h, s_)
                if not COMPUTE_ONLY:
                    hops[h][1][s_].wait_recv()
                    if h + 1 < n_hops:
                        hops[h + 1][1][s_].start()
                if not COMM_ONLY:
                    process_sub(1, kvbufB, h, s_)

        wo16 = wo_ref[...].astype(BF16)
        if COMM_ONLY:
            for b in range(B):
                out_ref[b, :, :] = jnp.zeros((Sq_l, Dm), jnp.float32)
        for b in range(B if not COMM_ONLY else 0):
            ctx_blocks = [None] * n_blk
            for r in range(N_RES):
                l, acc = state[b][r]
                ctx3 = acc / l
                ctx_r = ctx3.transpose(1, 0, 2).reshape(R, HD)
                blocks = [r + N_RES * j for j in range(blk_per_res)]
                for j, rb in enumerate(blocks):
                    ctx_blocks[rb] = ctx_r[j * BLK:(j + 1) * BLK]
            ctx_b = jnp.concatenate(ctx_blocks, axis=0)
            out_ref[b, :, :] = jnp.dot(
                ctx_b.astype(BF16), wo16,
                preferred_element_type=jnp.float32)

        for hop in hops:
            for dir_subs in hop:
                for r_ in dir_subs:
                    r_.wait_send()

    kv = (2, 2, Skv_l // 2, Hq * Dh)
    return pl.pallas_call(
        body,
        out_shape=jax.ShapeDtypeStruct((B, Sq_l, Dm), jnp.float32),
        in_specs=[pl.BlockSpec(memory_space=pltpu.VMEM)] * 5,
        out_specs=pl.BlockSpec(memory_space=pltpu.VMEM),
        scratch_shapes=[
            pltpu.VMEM(kv, jnp.int8),
            pltpu.VMEM(kv, jnp.int8),
            pltpu.VMEM((n_hops,) + kv, jnp.int8),
            pltpu.VMEM((n_hops,) + kv, jnp.int8),
            pltpu.SemaphoreType.DMA((n_hops, 2)),
            pltpu.SemaphoreType.DMA((n_hops, 2)),
            pltpu.SemaphoreType.DMA((n_hops, 2)),
            pltpu.SemaphoreType.DMA((n_hops, 2)),
        ],
        compiler_params=pltpu.CompilerParams(
            collective_id=0, vmem_limit_bytes=100 * 1024 * 1024),
    )(x, Wq, K_ext, V_ext, Wo)


# device time: 51063 ns/iter; 1.4355x vs baseline; 1.4355x over previous
import jax
import jax.numpy as jnp
from jax import lax
from jax.experimental import pallas as pl
from jax.experimental.pallas import tpu as pltpu

N_DEV = 4
BLK = 64
N_RES = 4
BF16 = jnp.bfloat16
QSCALE = 127.0 / 6.0

COMPUTE_ONLY = False
COMM_ONLY = False


def kernel(x, Wq, K_ext, V_ext, Wo):
    B, Sq_l, Dm = x.shape
    _, Skv_l, Hq, Dh = K_ext.shape
    HD = Hq * Dh
    n_blk = Sq_l // BLK
    blk_per_res = n_blk // N_RES
    R = blk_per_res * BLK
    n_hops = N_DEV - 1
    scale = 1.0 / (Dh ** 0.5)

    def quant(mat):
        return jnp.round(
            jnp.clip(mat * QSCALE, -127.0, 127.0)).astype(jnp.int8)

    def dequant(mat):
        return mat.astype(BF16) * BF16(1.0 / QSCALE)

    def res_rows(mat):
        out = []
        for r in range(N_RES):
            blocks = [r + N_RES * j for j in range(blk_per_res)]
            out.append(jnp.concatenate(
                [mat[rb * BLK:(rb + 1) * BLK] for rb in blocks], axis=0))
        return out

    def regroup(mat):
        return jnp.concatenate(res_rows(mat), axis=0)

    def to_heads(mat):
        return mat.reshape(mat.shape[0], Hq, Dh)

    def body(x_ref, wq_ref, k_ref, v_ref, wo_ref, out_ref,
             kvownA, kvownB, kvbufA, kvbufB,
             sA, rA, sB, rB):
        my = lax.axis_index("i")
        left = (my - 1) % N_DEV
        right = (my + 1) % N_DEV

        barrier_sem = pltpu.get_barrier_semaphore()
        for nbr in (left, right):
            pl.semaphore_signal(
                barrier_sem, inc=1,
                device_id=(nbr,), device_id_type=pl.DeviceIdType.MESH,
            )
        pl.semaphore_wait(barrier_sem, 2)

        kvownA[0] = quant(regroup(k_ref[0].reshape(Skv_l, HD)))
        kvownA[1] = quant(regroup(v_ref[0].reshape(Skv_l, HD)))

        def make_hop(h):
            common = dict(device_id_type=pl.DeviceIdType.MESH)
            ra = pltpu.make_async_remote_copy(
                src_ref=kvownA if h == 0 else kvbufA.at[h - 1],
                dst_ref=kvbufA.at[h], send_sem=sA.at[h], recv_sem=rA.at[h],
                device_id=(right,), **common)
            rb_ = pltpu.make_async_remote_copy(
                src_ref=kvownB if h == 0 else kvbufB.at[h - 1],
                dst_ref=kvbufB.at[h], send_sem=sB.at[h], recv_sem=rB.at[h],
                device_id=(left,), **common)
            return (ra, rb_)

        if COMPUTE_ONLY:
            kvownB[0] = quant(regroup(k_ref[1].reshape(Skv_l, HD)))
            kvownB[1] = quant(regroup(v_ref[1].reshape(Skv_l, HD)))
            for h in range(n_hops):
                kvbufA[h] = kvownA[...]
                kvbufB[h] = kvownB[...]
            hops = []
        else:
            hops = [make_hop(h) for h in range(n_hops)]
            hops[0][0].start()
            kvownB[0] = quant(regroup(k_ref[1].reshape(Skv_l, HD)))
            kvownB[1] = quant(regroup(v_ref[1].reshape(Skv_l, HD)))
            hops[0][1].start()

        wq16 = wq_ref[...].astype(BF16)
        q16 = []
        for b in range(B):
            q_b = jnp.dot(x_ref[b].astype(BF16), wq16,
                          preferred_element_type=jnp.float32)
            q16.append([to_heads(qr.astype(BF16)) for qr in res_rows(q_b)])

        state = [[None] * N_RES for _ in range(B)]

        def process(b, k_rs, v_rs):
            for r in range(N_RES):
                q3 = q16[b][r]
                k3 = to_heads(k_rs[r])
                v3 = to_heads(v_rs[r])
                s = lax.dot_general(
                    q3, k3, (((2,), (2,)), ((1,), (1,))),
                    preferred_element_type=jnp.float32) * scale
                p = jnp.exp(s)
                l_c = jnp.sum(p, axis=-1, keepdims=True)
                acc_c = lax.dot_general(
                    p.astype(BF16), v3, (((2,), (0,)), ((0,), (1,))),
                    preferred_element_type=jnp.float32)
                st = state[b][r]
                state[b][r] = (
                    (l_c, acc_c) if st is None
                    else (st[0] + l_c, st[1] + acc_c))

        def grouped_slices(mat):
            return [mat[r * R:(r + 1) * R] for r in range(N_RES)]

        def process_buf(h):
            kvA = dequant(kvbufA[h])
            process(0, grouped_slices(kvA[0]), grouped_slices(kvA[1]))
            kvB = dequant(kvbufB[h])
            process(1, grouped_slices(kvB[0]), grouped_slices(kvB[1]))

        if not COMM_ONLY:
            process(0, res_rows(k_ref[0].reshape(Skv_l, HD).astype(BF16)),
                    res_rows(v_ref[0].reshape(Skv_l, HD).astype(BF16)))
            process(1, res_rows(k_ref[1].reshape(Skv_l, HD).astype(BF16)),
                    res_rows(v_ref[1].reshape(Skv_l, HD).astype(BF16)))

        for h in range(n_hops):
            if not COMPUTE_ONLY:
                for r_ in hops[h]:
                    r_.wait_recv()
                if h + 1 < n_hops:
                    for r_ in hops[h + 1]:
                        r_.start()
            if not COMM_ONLY:
                process_buf(h)

        wo16 = wo_ref[...].astype(BF16)
        if COMM_ONLY:
            for b in range(B):
                out_ref[b, :, :] = jnp.zeros((Sq_l, Dm), jnp.float32)
        for b in range(B if not COMM_ONLY else 0):
            ctx_blocks = [None] * n_blk
            for r in range(N_RES):
                l, acc = state[b][r]
                ctx3 = acc / l
                ctx_r = ctx3.transpose(1, 0, 2).reshape(R, HD)
                blocks = [r + N_RES * j for j in range(blk_per_res)]
                for j, rb in enumerate(blocks):
                    ctx_blocks[rb] = ctx_r[j * BLK:(j + 1) * BLK]
            ctx_b = jnp.concatenate(ctx_blocks, axis=0)
            out_ref[b, :, :] = jnp.dot(
                ctx_b.astype(BF16), wo16,
                preferred_element_type=jnp.float32)

        for hop in hops:
            for r_ in hop:
                r_.wait_send()

    kv = (2, Skv_l, Hq * Dh)
    return pl.pallas_call(
        body,
        out_shape=jax.ShapeDtypeStruct((B, Sq_l, Dm), jnp.float32),
        in_specs=[pl.BlockSpec(memory_space=pltpu.VMEM)] * 5,
        out_specs=pl.BlockSpec(memory_space=pltpu.VMEM),
        scratch_shapes=[
            pltpu.VMEM(kv, jnp.int8),
            pltpu.VMEM(kv, jnp.int8),
            pltpu.VMEM((n_hops,) + kv, jnp.int8),
            pltpu.VMEM((n_hops,) + kv, jnp.int8),
            pltpu.SemaphoreType.DMA((n_hops,)),
            pltpu.SemaphoreType.DMA((n_hops,)),
            pltpu.SemaphoreType.DMA((n_hops,)),
            pltpu.SemaphoreType.DMA((n_hops,)),
        ],
        compiler_params=pltpu.CompilerParams(
            collective_id=0, vmem_limit_bytes=100 * 1024 * 1024),
    )(x, Wq, K_ext, V_ext, Wo)
